# Initial kernel scaffold; baseline (speedup 1.0000x reference)
import jax
import jax.numpy as jnp
from jax.experimental import pallas as pl


def kernel(token_ids, W):
    raise NotImplementedError("write your pallas kernel here")


# speedup vs baseline: 1.7968x; 1.7968x over previous
import functools

import jax
import jax.numpy as jnp
from jax import lax
from jax.experimental import pallas as pl
from jax.experimental.pallas import tpu as pltpu
from jax.experimental.pallas import tpu_sc as plsc

_NC = 2
_NS = 16
_NW = _NC * _NS

_IDX_LANE = 128
_CHUNK_ROWS = 512
_GATHERS = _CHUNK_ROWS // _IDX_LANE


@functools.lru_cache(maxsize=None)
def _embed_lookup(n_rows, d):
    rows_per_w = n_rows // _NW
    n_chunks = rows_per_w // _CHUNK_ROWS
    idx_rows_per_w = rows_per_w // _IDX_LANE

    mesh = plsc.VectorSubcoreMesh(core_axis_name="c", subcore_axis_name="s")

    @functools.partial(
        pl.kernel,
        mesh=mesh,
        out_type=jax.ShapeDtypeStruct((n_rows, d), jnp.float32),
        scratch_types=[
            pltpu.VMEM((_GATHERS, _IDX_LANE), jnp.int32),
            pltpu.VMEM((_CHUNK_ROWS, d), jnp.float32),
            pltpu.SemaphoreType.DMA,
        ],
        compiler_params=pltpu.CompilerParams(use_tc_tiling_on_sc=False),
    )
    def k(table_hbm, idx_hbm, out_hbm, idx_v, rows_v, sem):
        wid = lax.axis_index("s") * _NC + lax.axis_index("c")
        idx_row0 = wid * idx_rows_per_w
        out_row0 = wid * rows_per_w

        def body(g, carry):
            pltpu.sync_copy(
                idx_hbm.at[pl.ds(idx_row0 + g * _GATHERS, _GATHERS)], idx_v)
            cps = []
            for j in range(_GATHERS):
                cps.append(pltpu.async_copy(
                    table_hbm.at[idx_v.at[j]],
                    rows_v.at[pl.ds(j * _IDX_LANE, _IDX_LANE)],
                    sem))
            for cp in cps:
                cp.wait()
            pltpu.sync_copy(
                rows_v, out_hbm.at[pl.ds(out_row0 + g * _CHUNK_ROWS, _CHUNK_ROWS)])
            return carry

        lax.fori_loop(0, n_chunks, body, 0)

    return k


def kernel(token_ids, W):
    b, h = token_ids.shape
    n = b * h
    d = W.shape[1]
    idx2d = token_ids.reshape(n // _IDX_LANE, _IDX_LANE).astype(jnp.int32)
    out = _embed_lookup(n, d)(W, idx2d)
    return out.reshape(b, h, d)
